# baseline (speedup 1.0000x reference)
import jax
import jax.numpy as jnp
from jax.experimental import pallas as pl

H = W = 64
N_NODES = H * W
IN_DIM = 12
HID = 64
N_LAYERS = 3
EDGE_DIM_ = 3
NEG = -1e30


def _gnn_kernel(x_ref, in_w_ref, in_b_ref, We_w_ref, We_b_ref, Wn_w_ref,
                Wn_b_ref, ln_g_ref, ln_b_ref, head_w_ref, head_b_ref,
                out_ref):
    bb = x_ref.shape[0]
    x4 = x_ref[:]

    h = jax.lax.dot_general(in_w_ref[:], x4, (((0,), (1,)), ((), ())),
                            preferred_element_type=jnp.float32)
    h = h + in_b_ref[:].reshape(HID, 1, 1)
    v0 = x4[:, 0, :][None]

    jl = jax.lax.broadcasted_iota(jnp.int32, (1, 1, N_NODES), 2) % W
    m_left = (jl != 0).astype(jnp.float32)
    m_right = (jl != W - 1).astype(jnp.float32)

    neg_row = jnp.full((HID, bb, W), NEG, jnp.float32)

    We_w = We_w_ref[:]
    We_b = We_b_ref[:]
    Wn_w = Wn_w_ref[:]
    Wn_b = Wn_b_ref[:]
    ln_g = ln_g_ref[:]
    ln_b = ln_b_ref[:]

    for l in range(N_LAYERS):
        w_dx = We_w[l, 0].reshape(HID, 1, 1)
        w_dy = We_w[l, 1].reshape(HID, 1, 1)
        w_vd = We_w[l, 2].reshape(HID, 1, 1)
        b_e = We_b[l].reshape(HID, 1, 1)

        t = v0 * w_vd
        g = h - t
        u_top = t + (b_e + w_dy)
        u_bot = t + (b_e - w_dy)
        u_left = t + (b_e + w_dx)
        u_right = t + (b_e - w_dx)

        g_top = jnp.concatenate([neg_row, g[..., :-W]], axis=2)
        g_bot = jnp.concatenate([g[..., W:], neg_row], axis=2)
        g_l = jnp.concatenate([neg_row[..., :1], g[..., :-1]], axis=2)
        g_r = jnp.concatenate([g[..., 1:], neg_row[..., :1]], axis=2)

        agg = (jnp.maximum(g_top + u_top, 0.0)
               + jnp.maximum(g_bot + u_bot, 0.0)
               + jnp.maximum(g_l + u_left, 0.0) * m_left
               + jnp.maximum(g_r + u_right, 0.0) * m_right)

        hn = jax.lax.dot_general(Wn_w[l], h + agg, (((0,), (0,)), ((), ())),
                                 preferred_element_type=jnp.float32)
        hn = hn + Wn_b[l].reshape(HID, 1, 1)
        mu = jnp.mean(hn, axis=0, keepdims=True)
        d = hn - mu
        var = jnp.mean(d * d, axis=0, keepdims=True)
        hn = (d * jax.lax.rsqrt(var + 1e-5)) * ln_g[l].reshape(HID, 1, 1) \
            + ln_b[l].reshape(HID, 1, 1)
        h = h + jnp.maximum(hn, 0.0)

    res = jax.lax.dot_general(head_w_ref[:], h, (((0,), (0,)), ((), ())),
                              preferred_element_type=jnp.float32)
    out_ref[:] = (res + head_b_ref[0, 0]).reshape(bb, 1, N_NODES)


BB = 1


def kernel(x, edge_index, edge_dirs, in_proj_w, in_proj_b, We_w, We_b, Wn_w,
           Wn_b, ln_g, ln_b, head_w, head_b, interpret=False):
    Bsz = x.shape[0]
    x2 = x.reshape(Bsz, IN_DIM, N_NODES)
    full = lambda s: pl.BlockSpec(s, lambda b: (0,) * len(s))
    out = pl.pallas_call(
        _gnn_kernel,
        grid=(Bsz // BB,),
        in_specs=[
            pl.BlockSpec((BB, IN_DIM, N_NODES), lambda b: (b, 0, 0)),
            full((IN_DIM, HID)),
            full((1, HID)),
            full((N_LAYERS, EDGE_DIM_, HID)),
            full((N_LAYERS, HID)),
            full((N_LAYERS, HID, HID)),
            full((N_LAYERS, HID)),
            full((N_LAYERS, HID)),
            full((N_LAYERS, HID)),
            full((HID, 1)),
            full((1, 1)),
        ],
        out_specs=pl.BlockSpec((BB, 1, N_NODES), lambda b: (b, 0, 0)),
        out_shape=jax.ShapeDtypeStruct((Bsz, 1, N_NODES), jnp.float32),
        interpret=interpret,
    )(x2, in_proj_w, in_proj_b.reshape(1, HID), We_w, We_b, Wn_w, Wn_b,
      ln_g, ln_b, head_w, head_b.reshape(1, 1))
    return out.reshape(Bsz, H, W)


# speedup vs baseline: 4.8596x; 4.8596x over previous
import jax
import jax.numpy as jnp
from jax.experimental import pallas as pl

H = W = 64
N_NODES = H * W
IN_DIM = 12
HID = 64
N_LAYERS = 3
EDGE_DIM_ = 3
NEG = -1e30
PAIR = 2


def _bdiag(w):
    z = jnp.zeros_like(w)
    return jnp.concatenate([jnp.concatenate([w, z], 1),
                            jnp.concatenate([z, w], 1)], 0)


def _tile2(v):
    return jnp.concatenate([v, v])[None, :]


def _gnn_kernel(x_ref, in_w_ref, in_b_ref, We_w_ref, We_b_ref, Wn_w_ref,
                Wn_b_ref, ln_g_ref, ln_b_ref, head_w_ref, head_b_ref,
                out_ref):
    xp = x_ref[:].reshape(PAIR * IN_DIM, N_NODES)

    in_w2 = _bdiag(in_w_ref[:])
    r = jax.lax.broadcasted_iota(jnp.int32, (PAIR * IN_DIM, PAIR * HID), 0)
    c = jax.lax.broadcasted_iota(jnp.int32, (PAIR * IN_DIM, PAIR * HID), 1)
    sel0 = ((r == 0) & (c < HID)) | ((r == IN_DIM) & (c >= HID))
    e0 = sel0.astype(jnp.float32)

    h = jax.lax.dot_general(xp, in_w2, (((0,), (0,)), ((), ())),
                            preferred_element_type=jnp.float32)
    v0 = jax.lax.dot_general(xp, e0, (((0,), (0,)), ((), ())),
                             preferred_element_type=jnp.float32)
    h = h + _tile2(in_b_ref[0])

    jrow = jax.lax.broadcasted_iota(jnp.int32, (N_NODES, 1), 0) % W
    m_left = (jrow != 0).astype(jnp.float32)
    m_right = (jrow != W - 1).astype(jnp.float32)

    neg_rows = jnp.full((W, PAIR * HID), NEG, jnp.float32)
    neg_row1 = neg_rows[:1]

    We_w = We_w_ref[:]
    We_b = We_b_ref[:]
    Wn_w = Wn_w_ref[:]
    Wn_b = Wn_b_ref[:]
    ln_g = ln_g_ref[:]
    ln_b = ln_b_ref[:]

    for l in range(N_LAYERS):
        w_dx = _tile2(We_w[l, 0])
        w_dy = _tile2(We_w[l, 1])
        w_vd = _tile2(We_w[l, 2])
        b_e = _tile2(We_b[l])

        t = v0 * w_vd
        g = h - t
        u_top = t + (b_e + w_dy)
        u_bot = t + (b_e - w_dy)
        u_left = t + (b_e + w_dx)
        u_right = t + (b_e - w_dx)

        g_top = jnp.concatenate([neg_rows, g[:-W]], axis=0)
        g_bot = jnp.concatenate([g[W:], neg_rows], axis=0)
        g_l = jnp.concatenate([neg_row1, g[:-1]], axis=0)
        g_r = jnp.concatenate([g[1:], neg_row1], axis=0)

        agg = (jnp.maximum(g_top + u_top, 0.0)
               + jnp.maximum(g_bot + u_bot, 0.0)
               + jnp.maximum(g_l + u_left, 0.0) * m_left
               + jnp.maximum(g_r + u_right, 0.0) * m_right)

        Wn_c = Wn_w[l] - jnp.mean(Wn_w[l], axis=1, keepdims=True)
        bn_c = Wn_b[l] - jnp.mean(Wn_b[l])
        d = jax.lax.dot_general(h + agg, _bdiag(Wn_c),
                                (((1,), (0,)), ((), ())),
                                preferred_element_type=jnp.float32)
        d = d + _tile2(bn_c)
        d2 = d * d
        var = jnp.concatenate([
            jnp.broadcast_to(jnp.mean(d2[:, :HID], 1, keepdims=True),
                             d2[:, :HID].shape),
            jnp.broadcast_to(jnp.mean(d2[:, HID:], 1, keepdims=True),
                             d2[:, HID:].shape)], 1)
        hn = (d * jax.lax.rsqrt(var + 1e-5)) * _tile2(ln_g[l]) \
            + _tile2(ln_b[l])
        h = h + jnp.maximum(hn, 0.0)

    head2 = _bdiag(head_w_ref[:])
    res = jax.lax.dot_general(head2, h, (((0,), (1,)), ((), ())),
                              preferred_element_type=jnp.float32)
    out_ref[:] = (res + head_b_ref[0, 0]).reshape(PAIR, 1, N_NODES)


def kernel(x, edge_index, edge_dirs, in_proj_w, in_proj_b, We_w, We_b, Wn_w,
           Wn_b, ln_g, ln_b, head_w, head_b, interpret=False):
    Bsz = x.shape[0]
    x2 = x.reshape(Bsz, IN_DIM, N_NODES)
    full = lambda s: pl.BlockSpec(s, lambda b: (0,) * len(s))
    out = pl.pallas_call(
        _gnn_kernel,
        grid=(Bsz // PAIR,),
        in_specs=[
            pl.BlockSpec((PAIR, IN_DIM, N_NODES), lambda b: (b, 0, 0)),
            full((IN_DIM, HID)),
            full((1, HID)),
            full((N_LAYERS, EDGE_DIM_, HID)),
            full((N_LAYERS, HID)),
            full((N_LAYERS, HID, HID)),
            full((N_LAYERS, HID)),
            full((N_LAYERS, HID)),
            full((N_LAYERS, HID)),
            full((HID, 1)),
            full((1, 1)),
        ],
        out_specs=pl.BlockSpec((PAIR, 1, N_NODES), lambda b: (b, 0, 0)),
        out_shape=jax.ShapeDtypeStruct((Bsz, 1, N_NODES), jnp.float32),
        interpret=interpret,
    )(x2, in_proj_w, in_proj_b.reshape(1, HID), We_w, We_b, Wn_w, Wn_b,
      ln_g, ln_b, head_w, head_b.reshape(1, 1))
    return out.reshape(Bsz, H, W)


# speedup vs baseline: 5.0026x; 1.0294x over previous
import jax
import jax.numpy as jnp
from jax.experimental import pallas as pl
from jax.experimental.pallas import tpu as pltpu

H = W = 64
N_NODES = H * W
IN_DIM = 12
HID = 64
N_LAYERS = 3
NEG = -1e30
PAIR = 2
H2 = PAIR * HID


def _gnn_kernel(x_ref, in_w2_ref, e0_ref, in_b2_ref, w_vd2_ref, u_c_ref,
                Wc2_ref, bnc2_ref, ln_g2_ref, ln_b2_ref, head2_ref,
                head_b_ref, out_ref, g_scr):
    xp = x_ref[:].reshape(PAIR * IN_DIM, N_NODES)

    h = jax.lax.dot_general(xp, in_w2_ref[:], (((0,), (0,)), ((), ())),
                            preferred_element_type=jnp.float32)
    v0 = jax.lax.dot_general(xp, e0_ref[:], (((0,), (0,)), ((), ())),
                             preferred_element_type=jnp.float32)
    h = h + in_b2_ref[:]

    jrow = jax.lax.broadcasted_iota(jnp.int32, (N_NODES, 1), 0) % W
    m_left = (jrow != 0).astype(jnp.float32)
    m_right = (jrow != W - 1).astype(jnp.float32)

    g_scr[:W] = jnp.full((W, H2), NEG, jnp.float32)
    g_scr[W + N_NODES:] = jnp.full((W, H2), NEG, jnp.float32)

    for l in range(N_LAYERS):
        w_vd = w_vd2_ref[l]
        u_c = u_c_ref[l]

        t = v0 * w_vd
        g_scr[W:W + N_NODES] = h - t
        u_top = t + u_c[0:1]
        u_bot = t + u_c[1:2]
        u_left = t + u_c[2:3]
        u_right = t + u_c[3:4]

        g_top = g_scr[0:N_NODES]
        g_bot = g_scr[2 * W:2 * W + N_NODES]
        g_l = g_scr[W - 1:W - 1 + N_NODES]
        g_r = g_scr[W + 1:W + 1 + N_NODES]

        agg = (jnp.maximum(g_top + u_top, 0.0)
               + jnp.maximum(g_bot + u_bot, 0.0)
               + jnp.maximum(g_l + u_left, 0.0) * m_left
               + jnp.maximum(g_r + u_right, 0.0) * m_right)

        d = jax.lax.dot_general(h + agg, Wc2_ref[l], (((1,), (0,)), ((), ())),
                                preferred_element_type=jnp.float32)
        d = d + bnc2_ref[l]
        d2 = d * d
        var = jnp.concatenate([
            jnp.broadcast_to(jnp.mean(d2[:, :HID], 1, keepdims=True),
                             (N_NODES, HID)),
            jnp.broadcast_to(jnp.mean(d2[:, HID:], 1, keepdims=True),
                             (N_NODES, HID))], 1)
        hn = (d * jax.lax.rsqrt(var + 1e-5)) * ln_g2_ref[l] + ln_b2_ref[l]
        h = h + jnp.maximum(hn, 0.0)

    res = jax.lax.dot_general(head2_ref[:], h, (((0,), (1,)), ((), ())),
                              preferred_element_type=jnp.float32)
    out_ref[:] = (res + head_b_ref[0, 0]).reshape(PAIR, 1, N_NODES)


def _bdiag(w):
    z = jnp.zeros_like(w)
    return jnp.concatenate([jnp.concatenate([w, z], 1),
                            jnp.concatenate([z, w], 1)], 0)


def kernel(x, edge_index, edge_dirs, in_proj_w, in_proj_b, We_w, We_b, Wn_w,
           Wn_b, ln_g, ln_b, head_w, head_b, interpret=False):
    Bsz = x.shape[0]
    x2 = x.reshape(Bsz, IN_DIM, N_NODES)

    tile2 = lambda v: jnp.concatenate([v, v], axis=-1)
    in_w2 = _bdiag(in_proj_w)
    e0 = jnp.zeros((PAIR * IN_DIM, H2), jnp.float32)
    e0 = e0.at[0, :HID].set(1.0).at[IN_DIM, HID:].set(1.0)
    in_b2 = tile2(in_proj_b)[None, :]
    w_vd2 = tile2(We_w[:, 2])[:, None, :]
    u_c = jnp.stack([tile2(We_b + We_w[:, 1]),
                     tile2(We_b - We_w[:, 1]),
                     tile2(We_b + We_w[:, 0]),
                     tile2(We_b - We_w[:, 0])], axis=1)
    Wn_c = Wn_w - jnp.mean(Wn_w, axis=2, keepdims=True)
    Wc2 = jnp.stack([_bdiag(Wn_c[l]) for l in range(N_LAYERS)])
    bnc2 = tile2(Wn_b - jnp.mean(Wn_b, axis=1, keepdims=True))[:, None, :]
    ln_g2 = tile2(ln_g)[:, None, :]
    ln_b2 = tile2(ln_b)[:, None, :]
    head2 = _bdiag(head_w)

    full = lambda s: pl.BlockSpec(s, lambda b: (0,) * len(s))
    out = pl.pallas_call(
        _gnn_kernel,
        grid=(Bsz // PAIR,),
        in_specs=[
            pl.BlockSpec((PAIR, IN_DIM, N_NODES), lambda b: (b, 0, 0)),
            full((PAIR * IN_DIM, H2)),
            full((PAIR * IN_DIM, H2)),
            full((1, H2)),
            full((N_LAYERS, 1, H2)),
            full((N_LAYERS, 4, H2)),
            full((N_LAYERS, H2, H2)),
            full((N_LAYERS, 1, H2)),
            full((N_LAYERS, 1, H2)),
            full((N_LAYERS, 1, H2)),
            full((H2, PAIR)),
            full((1, 1)),
        ],
        out_specs=pl.BlockSpec((PAIR, 1, N_NODES), lambda b: (b, 0, 0)),
        out_shape=jax.ShapeDtypeStruct((Bsz, 1, N_NODES), jnp.float32),
        scratch_shapes=[pltpu.VMEM((N_NODES + 2 * W, H2), jnp.float32)],
        interpret=interpret,
    )(x2, in_w2, e0, in_b2, w_vd2, u_c, Wc2, bnc2, ln_g2, ln_b2, head2,
      head_b.reshape(1, 1))
    return out.reshape(Bsz, H, W)


# speedup vs baseline: 8.9835x; 1.7958x over previous
import jax
import jax.numpy as jnp
from jax.experimental import pallas as pl
from jax.experimental.pallas import tpu as pltpu

H = W = 64
N_NODES = H * W
IN_DIM = 12
HID = 64
N_LAYERS = 3
NEG = -1e30
PAIR = 2
H2 = PAIR * HID


def _gnn_kernel(x_ref, in_w2_ref, w_vd2_ref, u_c_ref, Wc2_ref, bnc2_ref,
                ln_g2_ref, ln_b2_ref, head2_ref, head_b_ref, mmean_ref,
                out_ref, g_scr):
    xp = x_ref[:].reshape(PAIR * IN_DIM, N_NODES)
    ones = jnp.ones((1, N_NODES), jnp.float32)
    xp1 = jnp.concatenate([xp, ones], axis=0)

    hv = jax.lax.dot_general(xp1, in_w2_ref[:], (((0,), (0,)), ((), ())),
                             preferred_element_type=jnp.float32)
    h = hv[:, :H2]
    v0 = hv[:, H2:]

    jrow = jax.lax.broadcasted_iota(jnp.int32, (N_NODES, 1), 0) % W
    m_left = (jrow != 0).astype(jnp.float32)
    m_right = (jrow != W - 1).astype(jnp.float32)

    g_scr[:W] = jnp.full((W, H2), NEG, jnp.float32)
    g_scr[W + N_NODES:] = jnp.full((W, H2), NEG, jnp.float32)

    for l in range(N_LAYERS):
        w_vd = w_vd2_ref[l]
        u_c = u_c_ref[l]

        t = v0 * w_vd
        g_scr[W:W + N_NODES] = h - t

        g_top = g_scr[0:N_NODES]
        g_bot = g_scr[2 * W:2 * W + N_NODES]
        g_l = g_scr[W - 1:W - 1 + N_NODES]
        g_r = g_scr[W + 1:W + 1 + N_NODES]

        agg = (jnp.maximum(g_top + t + u_c[0:1], 0.0)
               + jnp.maximum(g_bot + t + u_c[1:2], 0.0)
               + jnp.maximum(g_l + t + u_c[2:3], 0.0) * m_left
               + jnp.maximum(g_r + t + u_c[3:4], 0.0) * m_right)

        s = h + agg
        Wc2 = Wc2_ref[l]
        d = jnp.concatenate(
            [jax.lax.dot_general(s[:, k:k + 128], Wc2,
                                 (((1,), (0,)), ((), ())),
                                 preferred_element_type=jnp.float32)
             for k in range(0, H2, 128)], axis=1)
        d = d + bnc2_ref[l]
        d2 = d * d
        var = jax.lax.dot_general(d2, mmean_ref[:], (((1,), (0,)), ((), ())),
                                  preferred_element_type=jnp.float32)
        hn = (d * jax.lax.rsqrt(var + 1e-5)) * ln_g2_ref[l] + ln_b2_ref[l]
        h = h + jnp.maximum(hn, 0.0)

    res = jax.lax.dot_general(head2_ref[:], h, (((0,), (1,)), ((), ())),
                              preferred_element_type=jnp.float32)
    out_ref[:] = (res + head_b_ref[0, 0]).reshape(PAIR, 1, N_NODES)


def _bdiag(w, n=2):
    rows = []
    z = jnp.zeros_like(w)
    for i in range(n):
        rows.append(jnp.concatenate([w if j == i else z for j in range(n)], 1))
    return jnp.concatenate(rows, 0)


def kernel(x, edge_index, edge_dirs, in_proj_w, in_proj_b, We_w, We_b, Wn_w,
           Wn_b, ln_g, ln_b, head_w, head_b, interpret=False):
    Bsz = x.shape[0]
    x2 = x.reshape(Bsz, IN_DIM, N_NODES)

    tile = lambda v: jnp.concatenate([v] * PAIR, axis=-1)
    e0 = jnp.zeros((PAIR * IN_DIM, H2), jnp.float32)
    for g in range(PAIR):
        e0 = e0.at[g * IN_DIM, g * HID:(g + 1) * HID].set(1.0)
    in_wb = jnp.concatenate([_bdiag(in_proj_w, PAIR),
                             tile(in_proj_b)[None, :]], 0)
    e0b = jnp.concatenate([e0, jnp.zeros((1, H2), jnp.float32)], 0)
    in_w2 = jnp.concatenate([in_wb, e0b], 1)
    w_vd2 = tile(We_w[:, 2])[:, None, :]
    u_c = jnp.stack([tile(We_b + We_w[:, 1]),
                     tile(We_b - We_w[:, 1]),
                     tile(We_b + We_w[:, 0]),
                     tile(We_b - We_w[:, 0])], axis=1)
    Wn_c = Wn_w - jnp.mean(Wn_w, axis=2, keepdims=True)
    Wc2 = jnp.stack([_bdiag(Wn_c[l], 2) for l in range(N_LAYERS)])
    bnc2 = tile(Wn_b - jnp.mean(Wn_b, axis=1, keepdims=True))[:, None, :]
    ln_g2 = tile(ln_g)[:, None, :]
    ln_b2 = tile(ln_b)[:, None, :]
    head2 = _bdiag(head_w, PAIR)
    mmean = _bdiag(jnp.full((HID, HID), 1.0 / HID, jnp.float32), PAIR)

    full = lambda s: pl.BlockSpec(s, lambda b: (0,) * len(s))
    out = pl.pallas_call(
        _gnn_kernel,
        grid=(Bsz // PAIR,),
        in_specs=[
            pl.BlockSpec((PAIR, IN_DIM, N_NODES), lambda b: (b, 0, 0)),
            full((PAIR * IN_DIM + 1, 2 * H2)),
            full((N_LAYERS, 1, H2)),
            full((N_LAYERS, 4, H2)),
            full((N_LAYERS, 128, 128)),
            full((N_LAYERS, 1, H2)),
            full((N_LAYERS, 1, H2)),
            full((N_LAYERS, 1, H2)),
            full((H2, PAIR)),
            full((1, 1)),
            full((H2, H2)),
        ],
        out_specs=pl.BlockSpec((PAIR, 1, N_NODES), lambda b: (b, 0, 0)),
        out_shape=jax.ShapeDtypeStruct((Bsz, 1, N_NODES), jnp.float32),
        scratch_shapes=[pltpu.VMEM((N_NODES + 2 * W, H2), jnp.float32)],
        interpret=interpret,
    )(x2, in_w2, w_vd2, u_c, Wc2, bnc2, ln_g2, ln_b2, head2,
      head_b.reshape(1, 1), mmean)
    return out.reshape(Bsz, H, W)
